# baseline (speedup 1.0000x reference)
import functools

import jax
import jax.numpy as jnp
from jax import lax
from jax.experimental import pallas as pl
from jax.experimental.pallas import tpu as pltpu
from jax.experimental.pallas import tpu_sc as plsc

NUM_ENTITIES = 1000000
EMBEDDING_DIM = 64
BATCH = 16384

_NC = 2
_NS = 16
_NW = _NC * _NS
_B_PER_W = BATCH // _NW
_CHUNK = 128
_NCH = _B_PER_W // _CHUNK


@functools.partial(
    pl.kernel,
    out_type=jax.ShapeDtypeStruct((BATCH, EMBEDDING_DIM), jnp.float32),
    mesh=plsc.VectorSubcoreMesh(core_axis_name="c", subcore_axis_name="s"),
    compiler_params=pltpu.CompilerParams(use_tc_tiling_on_sc=False),
    scratch_types=[
        pltpu.VMEM((_NCH, _CHUNK), jnp.int32),
        pltpu.VMEM((_CHUNK, EMBEDDING_DIM), jnp.float32),
        pltpu.VMEM((_CHUNK, EMBEDDING_DIM), jnp.float32),
        pltpu.SemaphoreType.DMA,
        pltpu.SemaphoreType.DMA,
        pltpu.SemaphoreType.DMA,
        pltpu.SemaphoreType.DMA,
    ],
)
def _gather_kernel(idx_hbm, table_hbm, out_hbm, idx_v, rows_a, rows_b,
                   gsem_a, gsem_b, wsem_a, wsem_b):
    wid = lax.axis_index("s") * _NC + lax.axis_index("c")
    base = wid * _B_PER_W
    pltpu.sync_copy(idx_hbm.at[wid], idx_v)

    bufs = (rows_a, rows_b)
    gsems = (gsem_a, gsem_b)
    wsems = (wsem_a, wsem_b)
    gathers = [None] * _NCH
    writes = [None] * _NCH

    gathers[0] = pltpu.async_copy(table_hbm.at[idx_v.at[0]], bufs[0], gsems[0])
    for j in range(_NCH):
        nxt = j + 1
        if nxt < _NCH:
            if nxt >= 2:
                writes[nxt - 2].wait()
            gathers[nxt] = pltpu.async_copy(
                table_hbm.at[idx_v.at[nxt]], bufs[nxt % 2], gsems[nxt % 2])
        gathers[j].wait()
        writes[j] = pltpu.async_copy(
            bufs[j % 2], out_hbm.at[pl.ds(base + j * _CHUNK, _CHUNK)],
            wsems[j % 2])
    writes[_NCH - 2].wait()
    writes[_NCH - 1].wait()


def kernel(entities, entity_embeddings):
    idx = entities.astype(jnp.int32).reshape(_NW, _NCH, _CHUNK)
    return _gather_kernel(idx, entity_embeddings)


# speedup vs baseline: 2.2240x; 2.2240x over previous
import functools

import jax
import jax.numpy as jnp
from jax import lax
from jax.experimental import pallas as pl
from jax.experimental.pallas import tpu as pltpu
from jax.experimental.pallas import tpu_sc as plsc

NUM_ENTITIES = 1000000
EMBEDDING_DIM = 64
BATCH = 16384

_NC = 2
_NS = 16
_NW = _NC * _NS
_B_PER_W = BATCH // _NW
_BLK = 128
_NBLK = _B_PER_W // _BLK
_WAVE = 8
_L = 16


@functools.partial(
    pl.kernel,
    out_type=jax.ShapeDtypeStruct((EMBEDDING_DIM, BATCH), jnp.float32),
    mesh=plsc.VectorSubcoreMesh(core_axis_name="c", subcore_axis_name="s"),
    compiler_params=pltpu.CompilerParams(needs_layout_passes=False),
    scratch_types=[
        pltpu.VMEM((_B_PER_W,), jnp.int32),
        pltpu.VMEM((_WAVE, EMBEDDING_DIM, 128), jnp.float32),
        pltpu.VMEM((EMBEDDING_DIM, _BLK), jnp.float32),
        pltpu.SemaphoreType.DMA,
        pltpu.SemaphoreType.DMA,
    ],
)
def _gather_kernel(idx_hbm, table_hbm, out_hbm, idx_v, ring_v, stage_v,
                   gsem, wsem):
    wid = lax.axis_index("s") * _NC + lax.axis_index("c")
    base = wid * _B_PER_W
    pltpu.sync_copy(idx_hbm.at[pl.ds(base, _B_PER_W)], idx_v)

    lane = lax.iota(jnp.int32, _L)

    def scalar_idx(i):
        v = plsc.load_gather(idx_v, [jnp.full((_L,), i, jnp.int32)])
        return lax.reduce_max(v, axes=(0,))

    def bcast(x):
        return lax.broadcast_in_dim(x, (_L,), ())

    for b in range(_NBLK):
        if b > 0:
            pltpu.make_async_copy(
                out_hbm.at[pl.ds(0, EMBEDDING_DIM), pl.ds(0, _BLK)],
                stage_v, wsem).wait()

        def wave(w, _):
            j0 = b * _BLK + w * _WAVE
            for j in range(_WAVE):
                r = scalar_idx(j0 + j)
                start = pl.multiple_of(
                    lax.shift_right_logical(r, 7) * 128, 128)
                pltpu.async_copy(
                    table_hbm.at[pl.ds(0, EMBEDDING_DIM), pl.ds(start, 128)],
                    ring_v.at[j], gsem)
            for j in range(_WAVE):
                pltpu.make_async_copy(
                    table_hbm.at[pl.ds(0, EMBEDDING_DIM), pl.ds(0, 128)],
                    ring_v.at[j], gsem).wait()
            for j in range(_WAVE):
                r = scalar_idx(j0 + j)
                col = bcast(lax.rem(r, jnp.int32(128)))
                ocol = bcast(w * _WAVE + j)
                for k in range(EMBEDDING_DIM // _L):
                    rows = k * _L + lane
                    v = plsc.load_gather(
                        ring_v, [jnp.full((_L,), j, jnp.int32), rows, col])
                    plsc.store_scatter(stage_v, [rows, ocol], v)
            return ()

        lax.fori_loop(0, _BLK // _WAVE, wave, ())
        pltpu.async_copy(
            stage_v,
            out_hbm.at[pl.ds(0, EMBEDDING_DIM),
                       pl.ds(pl.multiple_of(base + b * _BLK, _BLK), _BLK)],
            wsem)
    pltpu.make_async_copy(
        out_hbm.at[pl.ds(0, EMBEDDING_DIM), pl.ds(0, _BLK)],
        stage_v, wsem).wait()


def kernel(entities, entity_embeddings):
    idx = entities.astype(jnp.int32)
    out_t = _gather_kernel(idx, entity_embeddings.T)
    return out_t.T


# speedup vs baseline: 2.4151x; 1.0859x over previous
import functools

import jax
import jax.numpy as jnp
from jax import lax
from jax.experimental import pallas as pl
from jax.experimental.pallas import tpu as pltpu
from jax.experimental.pallas import tpu_sc as plsc

NUM_ENTITIES = 1000000
EMBEDDING_DIM = 64
BATCH = 16384

_NC = 2
_NS = 16
_NW = _NC * _NS
_B_PER_W = BATCH // _NW
_WAVE = 11
_NWAVES = _B_PER_W // _WAVE
_TAIL = _B_PER_W - _NWAVES * _WAVE
_L = 16


@functools.partial(
    pl.kernel,
    out_type=jax.ShapeDtypeStruct((EMBEDDING_DIM, BATCH), jnp.float32),
    mesh=plsc.VectorSubcoreMesh(core_axis_name="c", subcore_axis_name="s"),
    compiler_params=pltpu.CompilerParams(needs_layout_passes=False),
    scratch_types=[
        pltpu.VMEM((_B_PER_W,), jnp.int32),
        pltpu.VMEM((_WAVE, EMBEDDING_DIM, 128), jnp.float32),
        pltpu.VMEM((EMBEDDING_DIM, _B_PER_W), jnp.float32),
        pltpu.SMEM((_B_PER_W,), jnp.int32),
        pltpu.SMEM((_B_PER_W,), jnp.int32),
        pltpu.SemaphoreType.DMA,
    ],
)
def _gather_kernel(idx_hbm, table_hbm, out_hbm, idx_v, ring_v, stage_v,
                   start_s, col_s, gsem):
    wid = lax.axis_index("s") * _NC + lax.axis_index("c")
    base = wid * _B_PER_W
    pltpu.sync_copy(idx_hbm.at[pl.ds(base, _B_PER_W)], idx_v)

    lane = lax.iota(jnp.int32, _L)

    def scalar_idx(i):
        v = plsc.load_gather(idx_v, [jnp.full((_L,), i, jnp.int32)])
        return lax.reduce_max(v, axes=(0,))

    def prep(i, _):
        r = scalar_idx(i)
        start_s[i] = lax.shift_right_logical(r, 7) * 128
        col_s[i] = lax.rem(r, jnp.int32(128))
        return ()

    lax.fori_loop(0, _B_PER_W, prep, ())

    def bcast(x):
        return lax.broadcast_in_dim(x, (_L,), ())

    def fire(t, s):
        st = pl.multiple_of(start_s[t], 128)
        pltpu.async_copy(
            table_hbm.at[pl.ds(0, EMBEDDING_DIM), pl.ds(st, 128)],
            ring_v.at[s], gsem)

    def drain(s):
        pltpu.make_async_copy(
            table_hbm.at[pl.ds(0, EMBEDDING_DIM), pl.ds(0, 128)],
            ring_v.at[s], gsem).wait()

    def extract(t, s):
        col = bcast(col_s[t])
        ocol = bcast(t)
        for k in range(EMBEDDING_DIM // _L):
            rows = k * _L + lane
            v = plsc.load_gather(
                ring_v, [jnp.full((_L,), s, jnp.int32), rows, col])
            plsc.store_scatter(stage_v, [rows, ocol], v)

    def wave(w, _):
        t0 = w * _WAVE
        for s in range(_WAVE):
            fire(t0 + s, s)
        for s in range(_WAVE):
            drain(s)
        for s in range(_WAVE):
            extract(t0 + s, s)
        return ()

    lax.fori_loop(0, _NWAVES, wave, ())

    t0 = _NWAVES * _WAVE
    for s in range(_TAIL):
        fire(t0 + s, s)
    for s in range(_TAIL):
        drain(s)
    for s in range(_TAIL):
        extract(t0 + s, s)

    pltpu.sync_copy(
        stage_v,
        out_hbm.at[pl.ds(0, EMBEDDING_DIM),
                   pl.ds(pl.multiple_of(base, _B_PER_W), _B_PER_W)])


def kernel(entities, entity_embeddings):
    idx = entities.astype(jnp.int32)
    out_t = _gather_kernel(idx, entity_embeddings.T)
    return out_t.T


# speedup vs baseline: 2.4378x; 1.0094x over previous
import functools

import jax
import jax.numpy as jnp
from jax import lax
from jax.experimental import pallas as pl
from jax.experimental.pallas import tpu as pltpu
from jax.experimental.pallas import tpu_sc as plsc

NUM_ENTITIES = 1000000
EMBEDDING_DIM = 64
BATCH = 16384

_NC = 2
_NS = 16
_NW = _NC * _NS
_B_PER_W = BATCH // _NW
_WAVE = 11
_NWAVES = _B_PER_W // _WAVE
_TAIL = _B_PER_W - _NWAVES * _WAVE
_L = 16


@functools.partial(
    pl.kernel,
    out_type=jax.ShapeDtypeStruct((EMBEDDING_DIM, BATCH), jnp.float32),
    mesh=plsc.VectorSubcoreMesh(core_axis_name="c", subcore_axis_name="s"),
    compiler_params=pltpu.CompilerParams(needs_layout_passes=False),
    scratch_types=[
        pltpu.VMEM((_B_PER_W,), jnp.int32),
        pltpu.VMEM((_WAVE, EMBEDDING_DIM, 128), jnp.float32),
        pltpu.VMEM((EMBEDDING_DIM, _B_PER_W), jnp.float32),
        pltpu.SMEM((_B_PER_W,), jnp.int32),
        pltpu.SMEM((_B_PER_W,), jnp.int32),
        pltpu.SemaphoreType.DMA,
    ],
)
def _gather_kernel(idx_hbm, table_hbm, out_hbm, idx_v, ring_v, stage_v,
                   start_s, col_s, gsem):
    wid = lax.axis_index("s") * _NC + lax.axis_index("c")
    base = wid * _B_PER_W
    pltpu.sync_copy(idx_hbm.at[pl.ds(base, _B_PER_W)], idx_v)

    lane = lax.iota(jnp.int32, _L)

    def prep(c, _):
        chunk = idx_v[pl.ds(c * _L, _L)]
        starts = lax.shift_right_logical(chunk, 7) * 128
        cols = lax.rem(chunk, jnp.int32(128))
        for j in range(_L):
            start_s[c * _L + j] = lax.squeeze(
                lax.slice(starts, (j,), (j + 1,)), (0,))
            col_s[c * _L + j] = lax.squeeze(
                lax.slice(cols, (j,), (j + 1,)), (0,))
        return ()

    lax.fori_loop(0, _B_PER_W // _L, prep, ())

    def bcast(x):
        return lax.broadcast_in_dim(x, (_L,), ())

    def fire(t, s):
        st = pl.multiple_of(start_s[t], 128)
        pltpu.async_copy(
            table_hbm.at[pl.ds(0, EMBEDDING_DIM), pl.ds(st, 128)],
            ring_v.at[s], gsem)

    def drain(s):
        pltpu.make_async_copy(
            table_hbm.at[pl.ds(0, EMBEDDING_DIM), pl.ds(0, 128)],
            ring_v.at[s], gsem).wait()

    def extract(t, s):
        col = bcast(col_s[t])
        ocol = bcast(t)
        for k in range(EMBEDDING_DIM // _L):
            rows = k * _L + lane
            v = plsc.load_gather(
                ring_v, [jnp.full((_L,), s, jnp.int32), rows, col])
            plsc.store_scatter(stage_v, [rows, ocol], v)

    def wave(w, _):
        t0 = w * _WAVE
        for s in range(_WAVE):
            fire(t0 + s, s)
        for s in range(_WAVE):
            drain(s)
        for s in range(_WAVE):
            extract(t0 + s, s)
        return ()

    lax.fori_loop(0, _NWAVES, wave, ())

    t0 = _NWAVES * _WAVE
    for s in range(_TAIL):
        fire(t0 + s, s)
    for s in range(_TAIL):
        drain(s)
    for s in range(_TAIL):
        extract(t0 + s, s)

    pltpu.sync_copy(
        stage_v,
        out_hbm.at[pl.ds(0, EMBEDDING_DIM),
                   pl.ds(pl.multiple_of(base, _B_PER_W), _B_PER_W)])


def kernel(entities, entity_embeddings):
    idx = entities.astype(jnp.int32)
    out_t = _gather_kernel(idx, entity_embeddings.T)
    return out_t.T
